# Initial kernel scaffold; baseline (speedup 1.0000x reference)
import jax
import jax.numpy as jnp
from jax.experimental import pallas as pl


def kernel(x, edge_index, W1, b1, W2, b2):
    raise NotImplementedError("write your pallas kernel here")


# speedup vs baseline: 23.9048x; 23.9048x over previous
import functools

import jax
import jax.numpy as jnp
from jax import lax
from jax.experimental import pallas as pl
from jax.experimental.pallas import tpu as pltpu
from jax.experimental.pallas import tpu_sc as plsc

N = 10000
E = 320000
NTILES = 32
KROWS = 79
EPAD = NTILES * KROWS * 128
NROWS = 10240
RPT = NROWS // 16
D1 = 48
D2 = 32
DD = 8

@functools.cache
def _get_mesh():
    return plsc.VectorSubcoreMesh(core_axis_name="c", subcore_axis_name="s")


@functools.cache
def _make_sc_agg(D):
    @functools.partial(
        pl.kernel,
        out_type=jax.ShapeDtypeStruct((2, NROWS, D), jnp.float32),
        mesh=_get_mesh(),
        compiler_params=pltpu.CompilerParams(use_tc_tiling_on_sc=False),
        scratch_types=[
            pltpu.VMEM((KROWS, 128), jnp.int32),
            pltpu.VMEM((KROWS, 128), jnp.int32),
            pltpu.VMEM((128, D), jnp.float32),
            pltpu.VMEM((RPT, D), jnp.float32),
            pltpu.VMEM_SHARED((NROWS, D), jnp.float32),
            pltpu.SemaphoreType.DMA,
        ],
    )
    def agg(u_hbm, src_hbm, dst_hbm, out_hbm, srcv, dstv, buf, stage, acc, sem):
        c = lax.axis_index("c")
        s = lax.axis_index("s")
        wid = s * 2 + c

        pltpu.sync_copy(src_hbm.at[wid], srcv)
        pltpu.sync_copy(dst_hbm.at[wid], dstv)

        zeros16 = jnp.zeros((16,), jnp.float32)

        @pl.loop(0, 128)
        def _(r):
            for col in range(D // 16):
                buf[r, pl.ds(col * 16, 16)] = zeros16

        @pl.loop(0, RPT // 128)
        def _(i):
            pltpu.sync_copy(buf, acc.at[pl.ds(s * RPT + i * 128, 128)])

        plsc.subcore_barrier()

        @pl.loop(0, KROWS)
        def _(j):
            pltpu.async_copy(u_hbm.at[srcv.at[j]], buf, sem).wait()
            pltpu.sync_copy(buf, acc.at[dstv.at[j]], add=True)

        plsc.subcore_barrier()

        pltpu.sync_copy(acc.at[pl.ds(s * RPT, RPT)], stage)
        pltpu.sync_copy(stage, out_hbm.at[c, pl.ds(s * RPT, RPT)])

    return agg


@functools.cache
def _make_sc_deg():

    @functools.partial(
        pl.kernel,
        out_type=jax.ShapeDtypeStruct((2, NROWS, DD), jnp.float32),
        mesh=_get_mesh(),
        compiler_params=pltpu.CompilerParams(use_tc_tiling_on_sc=False),
        scratch_types=[
            pltpu.VMEM((KROWS, 128), jnp.int32),
            pltpu.VMEM((128, DD), jnp.float32),
            pltpu.VMEM((128, DD), jnp.float32),
            pltpu.VMEM((RPT, DD), jnp.float32),
            pltpu.VMEM_SHARED((NROWS, DD), jnp.float32),
        ],
    )
    def deg(dst_hbm, ones_hbm, zero_hbm, out_hbm, dstv, ones, zbuf, stage, acc):
        c = lax.axis_index("c")
        s = lax.axis_index("s")
        wid = s * 2 + c

        pltpu.sync_copy(dst_hbm.at[wid], dstv)
        pltpu.sync_copy(ones_hbm, ones)
        pltpu.sync_copy(zero_hbm, zbuf)

        @pl.loop(0, RPT // 128)
        def _(i):
            pltpu.sync_copy(zbuf, acc.at[pl.ds(s * RPT + i * 128, 128)])

        plsc.subcore_barrier()

        @pl.loop(0, KROWS)
        def _(j):
            pltpu.sync_copy(ones, acc.at[dstv.at[j]], add=True)

        plsc.subcore_barrier()

        pltpu.sync_copy(acc.at[pl.ds(s * RPT, RPT)], stage)
        pltpu.sync_copy(stage, out_hbm.at[c, pl.ds(s * RPT, RPT)])

    return deg


def _mm1_body(x_ref, w_ref, o_ref):
    o_ref[...] = jnp.dot(x_ref[...], w_ref[...],
                         preferred_element_type=jnp.float32)


def _tc_mm1(x, w1p):
    return pl.pallas_call(
        _mm1_body,
        out_shape=jax.ShapeDtypeStruct((N, D1), jnp.float32),
    )(x, w1p)


def _scale_body(degp_ref, xw_ref, u_ref, dinv_ref):
    deg = degp_ref[0, :N, 0:1] + degp_ref[1, :N, 0:1] + 1.0
    dinv = lax.rsqrt(deg)
    dinv_ref[...] = dinv
    u_ref[...] = xw_ref[...] * dinv


def _tc_scale(degp, xw):
    return pl.pallas_call(
        _scale_body,
        out_shape=(jax.ShapeDtypeStruct((N, D1), jnp.float32),
                   jax.ShapeDtypeStruct((N, 1), jnp.float32)),
    )(degp, xw)


def _layer_body(aggp_ref, u_ref, dinv_ref, b1_ref, w2_ref, u2_ref):
    dinv = dinv_ref[...]
    a = aggp_ref[0, :N, :] + aggp_ref[1, :N, :] + u_ref[...]
    h = jnp.maximum(a * dinv + b1_ref[...], 0.0)
    u2_ref[...] = jnp.dot(h, w2_ref[...],
                          preferred_element_type=jnp.float32) * dinv


def _tc_layer(aggp, u1, dinv, b1p, w2p):
    return pl.pallas_call(
        _layer_body,
        out_shape=jax.ShapeDtypeStruct((N, D2), jnp.float32),
    )(aggp, u1, dinv, b1p, w2p)


def _final_body(aggp_ref, u2_ref, dinv_ref, b2_ref, o_ref):
    a = aggp_ref[0, :N, :] + aggp_ref[1, :N, :] + u2_ref[...]
    o_ref[...] = (a * dinv_ref[...] + b2_ref[...])[:, :20]


def _tc_final(aggp, u2, dinv, b2p):
    return pl.pallas_call(
        _final_body,
        out_shape=jax.ShapeDtypeStruct((N, 20), jnp.float32),
    )(aggp, u2, dinv, b2p)


def kernel(x, edge_index, W1, b1, W2, b2):
    src = edge_index[0].astype(jnp.int32)
    dst = edge_index[1].astype(jnp.int32)
    npad = EPAD - E
    srcp = jnp.concatenate(
        [src, jnp.zeros((npad,), jnp.int32)]).reshape(NTILES, KROWS, 128)
    dstp = jnp.concatenate(
        [dst, N + (jnp.arange(npad, dtype=jnp.int32) % (NROWS - N))]
    ).reshape(NTILES, KROWS, 128)

    w1p = jnp.pad(W1, ((0, 0), (0, D1 - 40)))
    b1p = jnp.pad(b1, (0, D1 - 40)).reshape(1, D1)
    w2p = jnp.pad(W2, ((0, D1 - 40), (0, D2 - 20)))
    b2p = jnp.pad(b2, (0, D2 - 20)).reshape(1, D2)

    xw = _tc_mm1(x, w1p)
    ones_c = jnp.ones((128, DD), jnp.float32)
    zero_c = jnp.zeros((128, DD), jnp.float32)
    degp = _make_sc_deg()(dstp, ones_c, zero_c)
    u1, dinv = _tc_scale(degp, xw)
    agg1 = _make_sc_agg(D1)(u1, srcp, dstp)
    u2 = _tc_layer(agg1, u1, dinv, b1p, w2p)
    agg2 = _make_sc_agg(D2)(u2, srcp, dstp)
    return _tc_final(agg2, u2, dinv, b2p)


# speedup vs baseline: 24.8513x; 1.0396x over previous
import functools

import jax
import jax.numpy as jnp
from jax import lax
from jax.experimental import pallas as pl
from jax.experimental.pallas import tpu as pltpu
from jax.experimental.pallas import tpu_sc as plsc

N = 10000
E = 320000
NTILES = 32
KROWS = 80
EPAD = NTILES * KROWS * 128
NROWS = 10240
RPT = NROWS // 16
D1 = 48
D2 = 32
DD = 8

@functools.cache
def _get_mesh():
    return plsc.VectorSubcoreMesh(core_axis_name="c", subcore_axis_name="s")


@functools.cache
def _make_sc_agg(D):
    @functools.partial(
        pl.kernel,
        out_type=jax.ShapeDtypeStruct((2, NROWS, D), jnp.float32),
        mesh=_get_mesh(),
        compiler_params=pltpu.CompilerParams(use_tc_tiling_on_sc=False),
        scratch_types=[
            pltpu.VMEM((KROWS, 128), jnp.int32),
            pltpu.VMEM((KROWS, 128), jnp.int32),
            pltpu.VMEM((128, D), jnp.float32),
            pltpu.VMEM((128, D), jnp.float32),
            pltpu.VMEM((RPT, D), jnp.float32),
            pltpu.VMEM_SHARED((NROWS, D), jnp.float32),
            pltpu.SemaphoreType.DMA,
            pltpu.SemaphoreType.DMA,
        ],
    )
    def agg(u_hbm, src_hbm, dst_hbm, out_hbm, srcv, dstv, bufa, bufb,
            stage, acc, sema, semb):
        c = lax.axis_index("c")
        s = lax.axis_index("s")
        wid = s * 2 + c

        pltpu.sync_copy(src_hbm.at[wid], srcv)
        pltpu.sync_copy(dst_hbm.at[wid], dstv)

        zeros16 = jnp.zeros((16,), jnp.float32)

        @pl.loop(0, 128)
        def _(r):
            for col in range(D // 16):
                bufa[r, pl.ds(col * 16, 16)] = zeros16

        @pl.loop(0, RPT // 128)
        def _(i):
            pltpu.sync_copy(bufa, acc.at[pl.ds(s * RPT + i * 128, 128)])

        plsc.subcore_barrier()

        npairs = KROWS // 2
        pltpu.async_copy(u_hbm.at[srcv.at[0]], bufa, sema)
        pltpu.async_copy(u_hbm.at[srcv.at[1]], bufb, semb)

        @pl.loop(0, npairs)
        def _(p):
            j = p * 2
            pltpu.make_async_copy(u_hbm.at[srcv.at[j]], bufa, sema).wait()
            pltpu.sync_copy(bufa, acc.at[dstv.at[j]], add=True)

            @pl.when(p < npairs - 1)
            def _():
                pltpu.async_copy(u_hbm.at[srcv.at[j + 2]], bufa, sema)

            pltpu.make_async_copy(u_hbm.at[srcv.at[j + 1]], bufb, semb).wait()
            pltpu.sync_copy(bufb, acc.at[dstv.at[j + 1]], add=True)

            @pl.when(p < npairs - 1)
            def _():
                pltpu.async_copy(u_hbm.at[srcv.at[j + 3]], bufb, semb)

        plsc.subcore_barrier()

        pltpu.sync_copy(acc.at[pl.ds(s * RPT, RPT)], stage)
        pltpu.sync_copy(stage, out_hbm.at[c, pl.ds(s * RPT, RPT)])

    return agg


@functools.cache
def _make_sc_deg():

    @functools.partial(
        pl.kernel,
        out_type=jax.ShapeDtypeStruct((2, NROWS, DD), jnp.float32),
        mesh=_get_mesh(),
        compiler_params=pltpu.CompilerParams(use_tc_tiling_on_sc=False),
        scratch_types=[
            pltpu.VMEM((KROWS, 128), jnp.int32),
            pltpu.VMEM((128, DD), jnp.float32),
            pltpu.VMEM((128, DD), jnp.float32),
            pltpu.VMEM((RPT, DD), jnp.float32),
            pltpu.VMEM_SHARED((NROWS, DD), jnp.float32),
        ],
    )
    def deg(dst_hbm, ones_hbm, zero_hbm, out_hbm, dstv, ones, zbuf, stage, acc):
        c = lax.axis_index("c")
        s = lax.axis_index("s")
        wid = s * 2 + c

        pltpu.sync_copy(dst_hbm.at[wid], dstv)
        pltpu.sync_copy(ones_hbm, ones)
        pltpu.sync_copy(zero_hbm, zbuf)

        @pl.loop(0, RPT // 128)
        def _(i):
            pltpu.sync_copy(zbuf, acc.at[pl.ds(s * RPT + i * 128, 128)])

        plsc.subcore_barrier()

        @pl.loop(0, KROWS)
        def _(j):
            pltpu.sync_copy(ones, acc.at[dstv.at[j]], add=True)

        plsc.subcore_barrier()

        pltpu.sync_copy(acc.at[pl.ds(s * RPT, RPT)], stage)
        pltpu.sync_copy(stage, out_hbm.at[c, pl.ds(s * RPT, RPT)])

    return deg


def _mm1_body(x_ref, w_ref, o_ref):
    o_ref[...] = jnp.dot(x_ref[...], w_ref[...],
                         preferred_element_type=jnp.float32)


def _tc_mm1(x, w1p):
    return pl.pallas_call(
        _mm1_body,
        out_shape=jax.ShapeDtypeStruct((N, D1), jnp.float32),
    )(x, w1p)


def _scale_body(degp_ref, xw_ref, u_ref, dinv_ref):
    deg = degp_ref[0, :N, 0:1] + degp_ref[1, :N, 0:1] + 1.0
    dinv = lax.rsqrt(deg)
    dinv_ref[...] = dinv
    u_ref[...] = xw_ref[...] * dinv


def _tc_scale(degp, xw):
    return pl.pallas_call(
        _scale_body,
        out_shape=(jax.ShapeDtypeStruct((N, D1), jnp.float32),
                   jax.ShapeDtypeStruct((N, 1), jnp.float32)),
    )(degp, xw)


def _layer_body(aggp_ref, u_ref, dinv_ref, b1_ref, w2_ref, u2_ref):
    dinv = dinv_ref[...]
    a = aggp_ref[0, :N, :] + aggp_ref[1, :N, :] + u_ref[...]
    h = jnp.maximum(a * dinv + b1_ref[...], 0.0)
    u2_ref[...] = jnp.dot(h, w2_ref[...],
                          preferred_element_type=jnp.float32) * dinv


def _tc_layer(aggp, u1, dinv, b1p, w2p):
    return pl.pallas_call(
        _layer_body,
        out_shape=jax.ShapeDtypeStruct((N, D2), jnp.float32),
    )(aggp, u1, dinv, b1p, w2p)


def _final_body(aggp_ref, u2_ref, dinv_ref, b2_ref, o_ref):
    a = aggp_ref[0, :N, :] + aggp_ref[1, :N, :] + u2_ref[...]
    o_ref[...] = (a * dinv_ref[...] + b2_ref[...])[:, :20]


def _tc_final(aggp, u2, dinv, b2p):
    return pl.pallas_call(
        _final_body,
        out_shape=jax.ShapeDtypeStruct((N, 20), jnp.float32),
    )(aggp, u2, dinv, b2p)


def kernel(x, edge_index, W1, b1, W2, b2):
    src = edge_index[0].astype(jnp.int32)
    dst = edge_index[1].astype(jnp.int32)
    npad = EPAD - E
    srcp = jnp.concatenate(
        [src, jnp.zeros((npad,), jnp.int32)]).reshape(NTILES, KROWS, 128)
    dstp = jnp.concatenate(
        [dst, N + (jnp.arange(npad, dtype=jnp.int32) % (NROWS - N))]
    ).reshape(NTILES, KROWS, 128)

    w1p = jnp.pad(W1, ((0, 0), (0, D1 - 40)))
    b1p = jnp.pad(b1, (0, D1 - 40)).reshape(1, D1)
    w2p = jnp.pad(W2, ((0, D1 - 40), (0, D2 - 20)))
    b2p = jnp.pad(b2, (0, D2 - 20)).reshape(1, D2)

    xw = _tc_mm1(x, w1p)
    ones_c = jnp.ones((128, DD), jnp.float32)
    zero_c = jnp.zeros((128, DD), jnp.float32)
    degp = _make_sc_deg()(dstp, ones_c, zero_c)
    u1, dinv = _tc_scale(degp, xw)
    agg1 = _make_sc_agg(D1)(u1, srcp, dstp)
    u2 = _tc_layer(agg1, u1, dinv, b1p, w2p)
    agg2 = _make_sc_agg(D2)(u2, srcp, dstp)
    return _tc_final(agg2, u2, dinv, b2p)


# speedup vs baseline: 26.8004x; 1.0784x over previous
import functools

import jax
import jax.numpy as jnp
from jax import lax
from jax.experimental import pallas as pl
from jax.experimental.pallas import tpu as pltpu
from jax.experimental.pallas import tpu_sc as plsc

N = 10000
E = 320000
NTILES = 32
KROWS = 80
EPAD = NTILES * KROWS * 128
NROWS = 10240
RPT = NROWS // 16
D1 = 40
D2 = 20
DD = 8

@functools.cache
def _get_mesh():
    return plsc.VectorSubcoreMesh(core_axis_name="c", subcore_axis_name="s")


@functools.cache
def _make_sc_agg(D):
    @functools.partial(
        pl.kernel,
        out_type=jax.ShapeDtypeStruct((2, NROWS, D), jnp.float32),
        mesh=_get_mesh(),
        compiler_params=pltpu.CompilerParams(use_tc_tiling_on_sc=False),
        scratch_types=[
            pltpu.VMEM((KROWS, 128), jnp.int32),
            pltpu.VMEM((KROWS, 128), jnp.int32),
            pltpu.VMEM((128, D), jnp.float32),
            pltpu.VMEM((128, D), jnp.float32),
            pltpu.VMEM((RPT, D), jnp.float32),
            pltpu.VMEM_SHARED((NROWS, D), jnp.float32),
            pltpu.SemaphoreType.DMA,
            pltpu.SemaphoreType.DMA,
        ],
    )
    def agg(u_hbm, src_hbm, dst_hbm, zero_hbm, out_hbm, srcv, dstv, bufa,
            bufb, stage, acc, sema, semb):
        c = lax.axis_index("c")
        s = lax.axis_index("s")
        wid = s * 2 + c

        pltpu.sync_copy(src_hbm.at[wid], srcv)
        pltpu.sync_copy(dst_hbm.at[wid], dstv)

        pltpu.sync_copy(zero_hbm, bufa)

        @pl.loop(0, RPT // 128)
        def _(i):
            pltpu.sync_copy(bufa, acc.at[pl.ds(s * RPT + i * 128, 128)])

        plsc.subcore_barrier()

        npairs = KROWS // 2
        pltpu.async_copy(u_hbm.at[srcv.at[0]], bufa, sema)
        pltpu.async_copy(u_hbm.at[srcv.at[1]], bufb, semb)

        @pl.loop(0, npairs)
        def _(p):
            j = p * 2
            pltpu.make_async_copy(u_hbm.at[srcv.at[j]], bufa, sema).wait()
            pltpu.sync_copy(bufa, acc.at[dstv.at[j]], add=True)

            @pl.when(p < npairs - 1)
            def _():
                pltpu.async_copy(u_hbm.at[srcv.at[j + 2]], bufa, sema)

            pltpu.make_async_copy(u_hbm.at[srcv.at[j + 1]], bufb, semb).wait()
            pltpu.sync_copy(bufb, acc.at[dstv.at[j + 1]], add=True)

            @pl.when(p < npairs - 1)
            def _():
                pltpu.async_copy(u_hbm.at[srcv.at[j + 3]], bufb, semb)

        plsc.subcore_barrier()

        pltpu.sync_copy(acc.at[pl.ds(s * RPT, RPT)], stage)
        pltpu.sync_copy(stage, out_hbm.at[c, pl.ds(s * RPT, RPT)])

    return agg


@functools.cache
def _make_sc_deg():

    @functools.partial(
        pl.kernel,
        out_type=jax.ShapeDtypeStruct((2, NROWS, DD), jnp.float32),
        mesh=_get_mesh(),
        compiler_params=pltpu.CompilerParams(use_tc_tiling_on_sc=False),
        scratch_types=[
            pltpu.VMEM((KROWS, 128), jnp.int32),
            pltpu.VMEM((128, DD), jnp.float32),
            pltpu.VMEM((128, DD), jnp.float32),
            pltpu.VMEM((RPT, DD), jnp.float32),
            pltpu.VMEM_SHARED((NROWS, DD), jnp.float32),
        ],
    )
    def deg(dst_hbm, ones_hbm, zero_hbm, out_hbm, dstv, ones, zbuf, stage, acc):
        c = lax.axis_index("c")
        s = lax.axis_index("s")
        wid = s * 2 + c

        pltpu.sync_copy(dst_hbm.at[wid], dstv)
        pltpu.sync_copy(ones_hbm, ones)
        pltpu.sync_copy(zero_hbm, zbuf)

        @pl.loop(0, RPT // 128)
        def _(i):
            pltpu.sync_copy(zbuf, acc.at[pl.ds(s * RPT + i * 128, 128)])

        plsc.subcore_barrier()

        @pl.loop(0, KROWS)
        def _(j):
            pltpu.sync_copy(ones, acc.at[dstv.at[j]], add=True)

        plsc.subcore_barrier()

        pltpu.sync_copy(acc.at[pl.ds(s * RPT, RPT)], stage)
        pltpu.sync_copy(stage, out_hbm.at[c, pl.ds(s * RPT, RPT)])

    return deg


def _mm1_body(x_ref, w_ref, o_ref):
    o_ref[...] = jnp.dot(x_ref[...], w_ref[...],
                         preferred_element_type=jnp.float32)


def _tc_mm1(x, w1p):
    return pl.pallas_call(
        _mm1_body,
        out_shape=jax.ShapeDtypeStruct((N, D1), jnp.float32),
    )(x, w1p)


def _scale_body(degp_ref, xw_ref, u_ref, dinv_ref):
    deg = degp_ref[0, :N, 0:1] + degp_ref[1, :N, 0:1] + 1.0
    dinv = lax.rsqrt(deg)
    dinv_ref[...] = dinv
    u_ref[...] = xw_ref[...] * dinv


def _tc_scale(degp, xw):
    return pl.pallas_call(
        _scale_body,
        out_shape=(jax.ShapeDtypeStruct((N, D1), jnp.float32),
                   jax.ShapeDtypeStruct((N, 1), jnp.float32)),
    )(degp, xw)


def _layer_body(aggp_ref, u_ref, dinv_ref, b1_ref, w2_ref, u2_ref):
    dinv = dinv_ref[...]
    a = aggp_ref[0, :N, :] + aggp_ref[1, :N, :] + u_ref[...]
    h = jnp.maximum(a * dinv + b1_ref[...], 0.0)
    u2_ref[...] = jnp.dot(h, w2_ref[...],
                          preferred_element_type=jnp.float32) * dinv


def _tc_layer(aggp, u1, dinv, b1p, w2p):
    return pl.pallas_call(
        _layer_body,
        out_shape=jax.ShapeDtypeStruct((N, D2), jnp.float32),
    )(aggp, u1, dinv, b1p, w2p)


def _final_body(aggp_ref, u2_ref, dinv_ref, b2_ref, o_ref):
    a = aggp_ref[0, :N, :] + aggp_ref[1, :N, :] + u2_ref[...]
    o_ref[...] = a * dinv_ref[...] + b2_ref[...]


def _tc_final(aggp, u2, dinv, b2p):
    return pl.pallas_call(
        _final_body,
        out_shape=jax.ShapeDtypeStruct((N, 20), jnp.float32),
    )(aggp, u2, dinv, b2p)


def kernel(x, edge_index, W1, b1, W2, b2):
    src = edge_index[0].astype(jnp.int32)
    dst = edge_index[1].astype(jnp.int32)
    npad = EPAD - E
    srcp = jnp.concatenate(
        [src, jnp.zeros((npad,), jnp.int32)]).reshape(NTILES, KROWS, 128)
    dstp = jnp.concatenate(
        [dst, N + (jnp.arange(npad, dtype=jnp.int32) % (NROWS - N))]
    ).reshape(NTILES, KROWS, 128)

    b1p = b1.reshape(1, D1)
    b2p = b2.reshape(1, D2)

    xw = _tc_mm1(x, W1)
    ones_c = jnp.ones((128, DD), jnp.float32)
    zero_c = jnp.zeros((128, DD), jnp.float32)
    zero1 = jnp.zeros((128, D1), jnp.float32)
    zero2 = jnp.zeros((128, D2), jnp.float32)
    degp = _make_sc_deg()(dstp, ones_c, zero_c)
    u1, dinv = _tc_scale(degp, xw)
    agg1 = _make_sc_agg(D1)(u1, srcp, dstp, zero1)
    u2 = _tc_layer(agg1, u1, dinv, b1p, W2)
    agg2 = _make_sc_agg(D2)(u2, srcp, dstp, zero2)
    return _tc_final(agg2, u2, dinv, b2p)
